# baseline (speedup 1.0000x reference)
import functools
import math

import jax
import jax.numpy as jnp
import numpy as np
from jax import lax
from jax.experimental import pallas as pl
from jax.experimental.pallas import tpu as pltpu
from jax.experimental.pallas import tpu_sc as plsc

_NC, _NS = 2, 16
_NW = _NC * _NS
_BPW = 6144 // _NW

_IN = 192
_GRID = 64
_P = _GRID * _GRID
_A = 9
_AL = 16
_K1 = 6000
_NPAD = 6144
_TOPK = 300
_MB = 512
_LOGM = math.log(1000.0 / 16.0)


def _anchor_planes(image_shape, feat_shape):
    grid_h, grid_w = feat_shape[-2], feat_shape[-1]
    image_h, image_w = image_shape[-2], image_shape[-1]
    stride_h = float(image_h // grid_h)
    stride_w = float(image_w // grid_w)
    scales = jnp.asarray((128.0, 256.0, 512.0), dtype=jnp.float32)
    aspect_ratios = jnp.asarray((0.5, 1.0, 2.0), dtype=jnp.float32)
    h_ratios = jnp.sqrt(aspect_ratios)
    w_ratios = 1.0 / h_ratios
    ws = (w_ratios[:, None] * scales[None, :]).reshape(-1)
    hs = (h_ratios[:, None] * scales[None, :]).reshape(-1)
    base = jnp.round(jnp.stack([-ws, -hs, ws, hs], axis=1) / 2.0)
    shifts_x = jnp.arange(0, grid_w, dtype=jnp.float32) * stride_w
    shifts_y = jnp.arange(0, grid_h, dtype=jnp.float32) * stride_h
    sy, sx = jnp.meshgrid(shifts_y, shifts_x, indexing="ij")
    sx = sx.reshape(-1)
    sy = sy.reshape(-1)
    shifts = jnp.stack((sx, sy, sx, sy), axis=1)
    anchors = (shifts[:, None, :] + base[None, :, :]).reshape(-1, 4)
    a = anchors.reshape(_P, _A, 4)
    aw = a[:, :, 2] - a[:, :, 0]
    ah = a[:, :, 3] - a[:, :, 1]
    acx = a[:, :, 0] + 0.5 * aw
    acy = a[:, :, 1] + 0.5 * ah
    one = jnp.ones((_P, _AL - _A), jnp.float32)
    pad = ((0, 0), (0, _AL - _A))
    aw = jnp.concatenate([aw, one], axis=1)
    ah = jnp.concatenate([ah, one], axis=1)
    acx = jnp.pad(acx, pad)
    acy = jnp.pad(acy, pad)
    return aw, ah, acx, acy


def _decode_kernel(im_h, im_w, d_ref, aw_ref, ah_ref, acx_ref, acy_ref,
                   sel_ref, out_ref):
    deltas = d_ref[...]
    pls = jnp.dot(deltas, sel_ref[...],
                  preferred_element_type=jnp.float32)
    dx = pls[:, 0:16]
    dy = pls[:, 16:32]
    dw = pls[:, 32:48]
    dh = pls[:, 48:64]
    aw = aw_ref[...]
    ah = ah_ref[...]
    pcx = dx * aw + acx_ref[...]
    pcy = dy * ah + acy_ref[...]
    pw = jnp.exp(jnp.minimum(dw, _LOGM)) * aw
    ph = jnp.exp(jnp.minimum(dh, _LOGM)) * ah
    x1 = jnp.clip(pcx - 0.5 * pw, 0.0, im_w)
    y1 = jnp.clip(pcy - 0.5 * ph, 0.0, im_h)
    x2 = jnp.clip(pcx + 0.5 * pw, 0.0, im_w)
    y2 = jnp.clip(pcy + 0.5 * ph, 0.0, im_h)
    ok = ((x2 - x1 >= 16.0) & (y2 - y1 >= 16.0)).astype(jnp.float32)
    pad = jnp.zeros((deltas.shape[0], 48), jnp.float32)
    out_ref[...] = jnp.concatenate([ok, x1, y1, x2, y2, pad], axis=1)


def _sc_gather_rows(table_hbm, idx_hbm, out_hbm, idx_v, rows_v, sem):
    wid = lax.axis_index("s") * _NC + lax.axis_index("c")
    base = wid * _BPW
    pltpu.sync_copy(idx_hbm.at[pl.ds(base, _BPW)], idx_v)
    pltpu.async_copy(table_hbm.at[idx_v], rows_v, sem).wait()
    pltpu.sync_copy(rows_v, out_hbm.at[pl.ds(base, _BPW)])


def _nms_kernel(b_ref, out_ref):
    x1 = b_ref[0:1, :]
    y1 = b_ref[1:2, :]
    x2 = b_ref[2:3, :]
    y2 = b_ref[3:4, :]
    sc = b_ref[4:5, :]
    area = (x2 - x1) * (y2 - y1)
    iota = lax.broadcasted_iota(jnp.int32, (1, _NPAD), 1)
    valid = sc > 0.0
    out_ref[...] = jnp.zeros((_TOPK + 4, 128), jnp.float32)
    li = lax.broadcasted_iota(jnp.int32, (1, 128), 1)

    def body(t, carry):
        excl, seld = carry
        availkey = jnp.where(valid & (excl == 0.0), sc, -2.0)
        m1 = jnp.max(availkey)
        has = m1 > 0.0
        idx1 = jnp.min(jnp.where(availkey == m1, iota, _NPAD))
        idx2 = jnp.min(jnp.where((seld == 0.0) & (iota < _K1), iota, _NPAD))
        bi = jnp.where(has, idx1, idx2)
        mb = iota == bi
        mbf = mb.astype(jnp.float32)
        bx1 = jnp.sum(x1 * mbf)
        by1 = jnp.sum(y1 * mbf)
        bx2 = jnp.sum(x2 * mbf)
        by2 = jnp.sum(y2 * mbf)
        bar = jnp.sum(area * mbf)
        iw = jnp.maximum(jnp.minimum(x2, bx2) - jnp.maximum(x1, bx1), 0.0)
        ih = jnp.maximum(jnp.minimum(y2, by2) - jnp.maximum(y1, by1), 0.0)
        inter = iw * ih
        iou = inter / (area + bar - inter)
        supf = (iou > 0.7).astype(jnp.float32)
        excl = jnp.maximum(excl, jnp.where(has, jnp.maximum(supf, mbf), mbf))
        seld = jnp.maximum(seld, mbf)
        outsc = jnp.where(has, m1, -jnp.inf)
        row = jnp.where(li == 0, bx1,
              jnp.where(li == 1, by1,
              jnp.where(li == 2, bx2,
              jnp.where(li == 3, by2,
              jnp.where(li == 4, outsc, 0.0)))))
        out_ref[pl.ds(t, 1), :] = row
        return excl, seld

    z = jnp.zeros((1, _NPAD), jnp.float32)
    lax.fori_loop(0, _TOPK, body, (z, z))


def kernel(image, feat, W_conv, b_conv, W_cls, b_cls, W_reg, b_reg):
    f32 = jnp.float32
    def conv2d(x, w, b, padding):
        out = lax.conv_general_dilated(x, w, window_strides=(1, 1),
                                       padding=padding,
                                       dimension_numbers=("NCHW", "OIHW", "NCHW"))
        return out + b[None, :, None, None]

    rpn = jax.nn.relu(conv2d(feat, W_conv, b_conv, "SAME"))
    cls = conv2d(rpn, W_cls, b_cls, "VALID")
    reg = conv2d(rpn, W_reg, b_reg, "VALID")
    score_flat = jax.nn.sigmoid(
        jnp.transpose(cls, (0, 2, 3, 1)).reshape(-1))
    deltas = jnp.transpose(reg[0].reshape(_A * 4, _P))
    deltas = jnp.pad(deltas, ((0, 0), (0, 64 - 4 * _A)))

    sel = np.zeros((64, 64), np.float32)
    for a in range(_A):
        for c in range(4):
            sel[4 * a + c, 16 * c + a] = 1.0
    sel = jnp.asarray(sel)
    aw, ah, acx, acy = _anchor_planes(image.shape, feat.shape)
    plane = lambda: pl.BlockSpec((_MB, _AL), lambda i: (i, 0))
    dec = pl.pallas_call(
        functools.partial(_decode_kernel, float(image.shape[-2]),
                          float(image.shape[-1])),
        grid=(_P // _MB,),
        in_specs=[
            pl.BlockSpec((_MB, 64), lambda i: (i, 0)),
            plane(), plane(), plane(), plane(),
            pl.BlockSpec((64, 64), lambda i: (0, 0)),
        ],
        out_specs=pl.BlockSpec((_MB, 128), lambda i: (i, 0)),
        out_shape=jax.ShapeDtypeStruct((_P, 128), f32),
    )(deltas, aw, ah, acx, acy, sel)

    lanes = jnp.arange(_AL)
    av = lanes[None, :] < _A
    score9 = score_flat.reshape(_P, _A)
    score16 = jnp.concatenate([score9, jnp.full((_P, _AL - _A), -1.0, f32)],
                              axis=1)
    ok16 = dec[:, 0:16]
    scf16 = jnp.where((ok16 > 0.0) & av, score16, -1.0)

    top_sc, top_idx = lax.top_k(score16.reshape(-1), _K1)
    scg = scf16.reshape(-1)[top_idx]
    order = jnp.argsort(-scg)
    idx_s = top_idx[order]
    sc_s = scg[order]
    padn = _NPAD - _K1
    table = jnp.pad(
        dec[:, 16:80].reshape(_P, 4, _AL).transpose(0, 2, 1).reshape(_P * _AL, 4),
        ((0, 0), (0, 124)))
    idx_pad = jnp.pad(idx_s, (0, padn)).astype(jnp.int32)
    mesh = plsc.VectorSubcoreMesh(core_axis_name="c", subcore_axis_name="s")
    rows = pl.kernel(
        _sc_gather_rows, mesh=mesh,
        out_type=jax.ShapeDtypeStruct((_NPAD, 128), f32),
        scratch_types=[
            pltpu.VMEM((_BPW,), jnp.int32),
            pltpu.VMEM((_BPW, 128), f32),
            pltpu.SemaphoreType.DMA,
        ])(table, idx_pad)
    nms_in = jnp.concatenate([
        rows[:, 0:4].T,
        jnp.pad(sc_s[None, :], ((0, 0), (0, padn)), constant_values=-1.0),
        jnp.zeros((3, _NPAD), f32)], axis=0)

    res = pl.pallas_call(
        _nms_kernel,
        out_shape=jax.ShapeDtypeStruct((_TOPK + 4, 128), f32),
    )(nms_in)
    return res[:_TOPK, 0:4], res[:_TOPK, 4]


# speedup vs baseline: 1.0571x; 1.0571x over previous
import functools
import math

import jax
import jax.numpy as jnp
import numpy as np
from jax import lax
from jax.experimental import pallas as pl
from jax.experimental.pallas import tpu as pltpu
from jax.experimental.pallas import tpu_sc as plsc

_NC, _NS = 2, 16
_NW = _NC * _NS
_BPW = 6144 // _NW

_IN = 192
_GRID = 64
_P = _GRID * _GRID
_A = 9
_AL = 16
_K1 = 6000
_NPAD = 6144
_TOPK = 300
_MB = 512
_LOGM = math.log(1000.0 / 16.0)


def _anchor_planes(image_shape, feat_shape):
    grid_h, grid_w = feat_shape[-2], feat_shape[-1]
    image_h, image_w = image_shape[-2], image_shape[-1]
    stride_h = float(image_h // grid_h)
    stride_w = float(image_w // grid_w)
    scales = jnp.asarray((128.0, 256.0, 512.0), dtype=jnp.float32)
    aspect_ratios = jnp.asarray((0.5, 1.0, 2.0), dtype=jnp.float32)
    h_ratios = jnp.sqrt(aspect_ratios)
    w_ratios = 1.0 / h_ratios
    ws = (w_ratios[:, None] * scales[None, :]).reshape(-1)
    hs = (h_ratios[:, None] * scales[None, :]).reshape(-1)
    base = jnp.round(jnp.stack([-ws, -hs, ws, hs], axis=1) / 2.0)
    shifts_x = jnp.arange(0, grid_w, dtype=jnp.float32) * stride_w
    shifts_y = jnp.arange(0, grid_h, dtype=jnp.float32) * stride_h
    sy, sx = jnp.meshgrid(shifts_y, shifts_x, indexing="ij")
    sx = sx.reshape(-1)
    sy = sy.reshape(-1)
    shifts = jnp.stack((sx, sy, sx, sy), axis=1)
    anchors = (shifts[:, None, :] + base[None, :, :]).reshape(-1, 4)
    a = anchors.reshape(_P, _A, 4)
    aw = a[:, :, 2] - a[:, :, 0]
    ah = a[:, :, 3] - a[:, :, 1]
    acx = a[:, :, 0] + 0.5 * aw
    acy = a[:, :, 1] + 0.5 * ah
    one = jnp.ones((_P, _AL - _A), jnp.float32)
    pad = ((0, 0), (0, _AL - _A))
    aw = jnp.concatenate([aw, one], axis=1)
    ah = jnp.concatenate([ah, one], axis=1)
    acx = jnp.pad(acx, pad)
    acy = jnp.pad(acy, pad)
    return aw, ah, acx, acy


def _decode_kernel(im_h, im_w, d_ref, aw_ref, ah_ref, acx_ref, acy_ref,
                   sel_ref, out_ref):
    deltas = d_ref[...]
    pls = jnp.dot(deltas, sel_ref[...],
                  preferred_element_type=jnp.float32)
    dx = pls[:, 0:16]
    dy = pls[:, 16:32]
    dw = pls[:, 32:48]
    dh = pls[:, 48:64]
    aw = aw_ref[...]
    ah = ah_ref[...]
    pcx = dx * aw + acx_ref[...]
    pcy = dy * ah + acy_ref[...]
    pw = jnp.exp(jnp.minimum(dw, _LOGM)) * aw
    ph = jnp.exp(jnp.minimum(dh, _LOGM)) * ah
    x1 = jnp.clip(pcx - 0.5 * pw, 0.0, im_w)
    y1 = jnp.clip(pcy - 0.5 * ph, 0.0, im_h)
    x2 = jnp.clip(pcx + 0.5 * pw, 0.0, im_w)
    y2 = jnp.clip(pcy + 0.5 * ph, 0.0, im_h)
    ok = ((x2 - x1 >= 16.0) & (y2 - y1 >= 16.0)).astype(jnp.float32)
    pad = jnp.zeros((deltas.shape[0], 48), jnp.float32)
    out_ref[...] = jnp.concatenate([ok, x1, y1, x2, y2, pad], axis=1)


def _sc_gather_rows(table_hbm, idx_hbm, out_hbm, idx_v, p_v, rows_v, sem):
    wid = lax.axis_index("s") * _NC + lax.axis_index("c")
    base = wid * _BPW
    pltpu.sync_copy(idx_hbm.at[pl.ds(base, _BPW)], idx_v)
    for t in range(_BPW // 16):
        v = idx_v[pl.ds(16 * t, 16)]
        p_v[pl.ds(16 * t, 16)] = jax.lax.shift_right_logical(v, 4)
    pltpu.async_copy(table_hbm.at[p_v], rows_v, sem).wait()
    pltpu.sync_copy(rows_v, out_hbm.at[pl.ds(base, _BPW)])


def _nms_kernel(b_ref, out_ref):
    x1 = b_ref[0:1, :]
    y1 = b_ref[1:2, :]
    x2 = b_ref[2:3, :]
    y2 = b_ref[3:4, :]
    sc = b_ref[4:5, :]
    area = (x2 - x1) * (y2 - y1)
    iota = lax.broadcasted_iota(jnp.int32, (1, _NPAD), 1)
    valid = sc > 0.0
    out_ref[...] = jnp.zeros((_TOPK + 4, 128), jnp.float32)
    li = lax.broadcasted_iota(jnp.int32, (1, 128), 1)

    def body(t, carry):
        excl, seld = carry
        availkey = jnp.where(valid & (excl == 0.0), sc, -2.0)
        m1 = jnp.max(availkey)
        has = m1 > 0.0
        idx1 = jnp.min(jnp.where(availkey == m1, iota, _NPAD))
        idx2 = jnp.min(jnp.where((seld == 0.0) & (iota < _K1), iota, _NPAD))
        bi = jnp.where(has, idx1, idx2)
        mb = iota == bi
        mbf = mb.astype(jnp.float32)
        bx1 = jnp.sum(x1 * mbf)
        by1 = jnp.sum(y1 * mbf)
        bx2 = jnp.sum(x2 * mbf)
        by2 = jnp.sum(y2 * mbf)
        bar = jnp.sum(area * mbf)
        iw = jnp.maximum(jnp.minimum(x2, bx2) - jnp.maximum(x1, bx1), 0.0)
        ih = jnp.maximum(jnp.minimum(y2, by2) - jnp.maximum(y1, by1), 0.0)
        inter = iw * ih
        iou = inter / (area + bar - inter)
        supf = (iou > 0.7).astype(jnp.float32)
        excl = jnp.maximum(excl, jnp.where(has, jnp.maximum(supf, mbf), mbf))
        seld = jnp.maximum(seld, mbf)
        outsc = jnp.where(has, m1, -jnp.inf)
        row = jnp.where(li == 0, bx1,
              jnp.where(li == 1, by1,
              jnp.where(li == 2, bx2,
              jnp.where(li == 3, by2,
              jnp.where(li == 4, outsc, 0.0)))))
        out_ref[pl.ds(t, 1), :] = row
        return excl, seld

    z = jnp.zeros((1, _NPAD), jnp.float32)
    lax.fori_loop(0, _TOPK, body, (z, z))


def kernel(image, feat, W_conv, b_conv, W_cls, b_cls, W_reg, b_reg):
    f32 = jnp.float32
    def conv2d(x, w, b, padding):
        out = lax.conv_general_dilated(x, w, window_strides=(1, 1),
                                       padding=padding,
                                       dimension_numbers=("NCHW", "OIHW", "NCHW"))
        return out + b[None, :, None, None]

    rpn = jax.nn.relu(conv2d(feat, W_conv, b_conv, "SAME"))
    cls = conv2d(rpn, W_cls, b_cls, "VALID")
    reg = conv2d(rpn, W_reg, b_reg, "VALID")
    score_flat = jax.nn.sigmoid(
        jnp.transpose(cls, (0, 2, 3, 1)).reshape(-1))
    deltas = jnp.transpose(reg[0].reshape(_A * 4, _P))
    deltas = jnp.pad(deltas, ((0, 0), (0, 64 - 4 * _A)))

    sel = np.zeros((64, 64), np.float32)
    for a in range(_A):
        for c in range(4):
            sel[4 * a + c, 16 * c + a] = 1.0
    sel = jnp.asarray(sel)
    aw, ah, acx, acy = _anchor_planes(image.shape, feat.shape)
    plane = lambda: pl.BlockSpec((_MB, _AL), lambda i: (i, 0))
    dec = pl.pallas_call(
        functools.partial(_decode_kernel, float(image.shape[-2]),
                          float(image.shape[-1])),
        grid=(_P // _MB,),
        in_specs=[
            pl.BlockSpec((_MB, 64), lambda i: (i, 0)),
            plane(), plane(), plane(), plane(),
            pl.BlockSpec((64, 64), lambda i: (0, 0)),
        ],
        out_specs=pl.BlockSpec((_MB, 128), lambda i: (i, 0)),
        out_shape=jax.ShapeDtypeStruct((_P, 128), f32),
    )(deltas, aw, ah, acx, acy, sel)

    lanes = jnp.arange(_AL)
    av = lanes[None, :] < _A
    score9 = score_flat.reshape(_P, _A)
    score16 = jnp.concatenate([score9, jnp.full((_P, _AL - _A), -1.0, f32)],
                              axis=1)
    ok16 = dec[:, 0:16]
    scf16 = jnp.where((ok16 > 0.0) & av, score16, -1.0)

    top_sc, top_idx = lax.top_k(score16.reshape(-1), _K1)
    scg = scf16.reshape(-1)[top_idx]
    order = jnp.argsort(-scg)
    idx_s = top_idx[order]
    sc_s = scg[order]
    padn = _NPAD - _K1
    idx_pad = jnp.pad(idx_s, (0, padn)).astype(jnp.int32)
    mesh = plsc.VectorSubcoreMesh(core_axis_name="c", subcore_axis_name="s")
    rows = pl.kernel(
        _sc_gather_rows, mesh=mesh,
        out_type=jax.ShapeDtypeStruct((_NPAD, 128), f32),
        scratch_types=[
            pltpu.VMEM((_BPW,), jnp.int32),
            pltpu.VMEM((_BPW,), jnp.int32),
            pltpu.VMEM((_BPW, 128), f32),
            pltpu.SemaphoreType.DMA,
        ])(dec, idx_pad)
    a_col = (idx_pad & 15)[:, None]
    box4 = jnp.concatenate(
        [jnp.take_along_axis(rows, 16 + 16 * c + a_col, axis=1)
         for c in range(4)], axis=1).T
    nms_in = jnp.concatenate([
        box4,
        jnp.pad(sc_s[None, :], ((0, 0), (0, padn)), constant_values=-1.0),
        jnp.zeros((3, _NPAD), f32)], axis=0)

    res = pl.pallas_call(
        _nms_kernel,
        out_shape=jax.ShapeDtypeStruct((_TOPK + 4, 128), f32),
    )(nms_in)
    return res[:_TOPK, 0:4], res[:_TOPK, 4]


# speedup vs baseline: 1.1451x; 1.0833x over previous
import functools
import math

import jax
import jax.numpy as jnp
import numpy as np
from jax import lax
from jax.experimental import pallas as pl
from jax.experimental.pallas import tpu as pltpu
from jax.experimental.pallas import tpu_sc as plsc

_NC, _NS = 2, 16
_NW = _NC * _NS
_BPW = 6144 // _NW

_IN = 192
_GRID = 64
_P = _GRID * _GRID
_A = 9
_AL = 16
_K1 = 6000
_NPAD = 6144
_TOPK = 300
_MB = 512
_LOGM = math.log(1000.0 / 16.0)


def _anchor_planes(image_shape, feat_shape):
    grid_h, grid_w = feat_shape[-2], feat_shape[-1]
    image_h, image_w = image_shape[-2], image_shape[-1]
    stride_h = float(image_h // grid_h)
    stride_w = float(image_w // grid_w)
    scales = jnp.asarray((128.0, 256.0, 512.0), dtype=jnp.float32)
    aspect_ratios = jnp.asarray((0.5, 1.0, 2.0), dtype=jnp.float32)
    h_ratios = jnp.sqrt(aspect_ratios)
    w_ratios = 1.0 / h_ratios
    ws = (w_ratios[:, None] * scales[None, :]).reshape(-1)
    hs = (h_ratios[:, None] * scales[None, :]).reshape(-1)
    base = jnp.round(jnp.stack([-ws, -hs, ws, hs], axis=1) / 2.0)
    shifts_x = jnp.arange(0, grid_w, dtype=jnp.float32) * stride_w
    shifts_y = jnp.arange(0, grid_h, dtype=jnp.float32) * stride_h
    sy, sx = jnp.meshgrid(shifts_y, shifts_x, indexing="ij")
    sx = sx.reshape(-1)
    sy = sy.reshape(-1)
    shifts = jnp.stack((sx, sy, sx, sy), axis=1)
    anchors = (shifts[:, None, :] + base[None, :, :]).reshape(-1, 4)
    a = anchors.reshape(_P, _A, 4)
    aw = a[:, :, 2] - a[:, :, 0]
    ah = a[:, :, 3] - a[:, :, 1]
    acx = a[:, :, 0] + 0.5 * aw
    acy = a[:, :, 1] + 0.5 * ah
    one = jnp.ones((_P, _AL - _A), jnp.float32)
    pad = ((0, 0), (0, _AL - _A))
    aw = jnp.concatenate([aw, one], axis=1)
    ah = jnp.concatenate([ah, one], axis=1)
    acx = jnp.pad(acx, pad)
    acy = jnp.pad(acy, pad)
    return aw, ah, acx, acy


def _decode_kernel(im_h, im_w, d_ref, aw_ref, ah_ref, acx_ref, acy_ref,
                   sel_ref, out_ref):
    deltas = d_ref[...]
    pls = jnp.dot(deltas, sel_ref[...],
                  preferred_element_type=jnp.float32)
    dx = pls[:, 0:16]
    dy = pls[:, 16:32]
    dw = pls[:, 32:48]
    dh = pls[:, 48:64]
    aw = aw_ref[...]
    ah = ah_ref[...]
    pcx = dx * aw + acx_ref[...]
    pcy = dy * ah + acy_ref[...]
    pw = jnp.exp(jnp.minimum(dw, _LOGM)) * aw
    ph = jnp.exp(jnp.minimum(dh, _LOGM)) * ah
    x1 = jnp.clip(pcx - 0.5 * pw, 0.0, im_w)
    y1 = jnp.clip(pcy - 0.5 * ph, 0.0, im_h)
    x2 = jnp.clip(pcx + 0.5 * pw, 0.0, im_w)
    y2 = jnp.clip(pcy + 0.5 * ph, 0.0, im_h)
    ok = ((x2 - x1 >= 16.0) & (y2 - y1 >= 16.0)).astype(jnp.float32)
    pad = jnp.zeros((deltas.shape[0], 48), jnp.float32)
    out_ref[...] = jnp.concatenate([ok, x1, y1, x2, y2, pad], axis=1)


def _sc_gather_rows(table_hbm, idx_hbm, out_hbm, idx_v, p_v, rows_v, sem):
    wid = lax.axis_index("s") * _NC + lax.axis_index("c")
    base = wid * _BPW
    pltpu.sync_copy(idx_hbm.at[pl.ds(base, _BPW)], idx_v)
    for t in range(_BPW // 16):
        v = idx_v[pl.ds(16 * t, 16)]
        p_v[pl.ds(16 * t, 16)] = jax.lax.shift_right_logical(v, 4)
    pltpu.async_copy(table_hbm.at[p_v], rows_v, sem).wait()
    pltpu.sync_copy(rows_v, out_hbm.at[pl.ds(base, _BPW)])


def _nms_kernel(b_ref, bt_ref, out_ref):
    x1 = b_ref[0:1, :]
    y1 = b_ref[1:2, :]
    x2 = b_ref[2:3, :]
    y2 = b_ref[3:4, :]
    sc = b_ref[4:5, :]
    area = (x2 - x1) * (y2 - y1)
    iota = lax.broadcasted_iota(jnp.int32, (1, _NPAD), 1)
    valid = sc > 0.0
    out_ref[...] = jnp.zeros((_TOPK + 4, 128), jnp.float32)
    li = lax.broadcasted_iota(jnp.int32, (1, 128), 1)

    def body(t, carry):
        excl, seld = carry
        availkey = jnp.where(valid & (excl == 0.0), sc, -2.0)
        m1 = jnp.max(availkey)
        has = m1 > 0.0
        bi = lax.cond(
            has,
            lambda: jnp.min(jnp.where(availkey == m1, iota, _NPAD)),
            lambda: jnp.min(jnp.where((seld == 0.0) & (iota < _K1), iota,
                                      _NPAD)))
        mb = iota == bi
        mbf = mb.astype(jnp.float32)
        blk = bt_ref[pl.ds((bi // 8) * 8, 8), :]
        si = lax.broadcasted_iota(jnp.int32, (8, 128), 0)
        l8 = lax.broadcasted_iota(jnp.int32, (8, 128), 1)
        hit = si == (bi % 8)
        bx1 = jnp.sum(jnp.where(hit & (l8 == 0), blk, 0.0))
        by1 = jnp.sum(jnp.where(hit & (l8 == 1), blk, 0.0))
        bx2 = jnp.sum(jnp.where(hit & (l8 == 2), blk, 0.0))
        by2 = jnp.sum(jnp.where(hit & (l8 == 3), blk, 0.0))
        bar = (bx2 - bx1) * (by2 - by1)
        iw = jnp.maximum(jnp.minimum(x2, bx2) - jnp.maximum(x1, bx1), 0.0)
        ih = jnp.maximum(jnp.minimum(y2, by2) - jnp.maximum(y1, by1), 0.0)
        inter = iw * ih
        iou = inter / (area + bar - inter)
        supf = (iou > 0.7).astype(jnp.float32)
        excl = jnp.maximum(excl, jnp.where(has, jnp.maximum(supf, mbf), mbf))
        seld = jnp.maximum(seld, mbf)
        outsc = jnp.where(has, m1, -jnp.inf)
        row = jnp.where(li == 0, bx1,
              jnp.where(li == 1, by1,
              jnp.where(li == 2, bx2,
              jnp.where(li == 3, by2,
              jnp.where(li == 4, outsc, 0.0)))))
        out_ref[pl.ds(t, 1), :] = row
        return excl, seld

    z = jnp.zeros((1, _NPAD), jnp.float32)
    lax.fori_loop(0, _TOPK, body, (z, z))


def kernel(image, feat, W_conv, b_conv, W_cls, b_cls, W_reg, b_reg):
    f32 = jnp.float32
    def conv2d(x, w, b, padding):
        out = lax.conv_general_dilated(x, w, window_strides=(1, 1),
                                       padding=padding,
                                       dimension_numbers=("NCHW", "OIHW", "NCHW"))
        return out + b[None, :, None, None]

    rpn = jax.nn.relu(conv2d(feat, W_conv, b_conv, "SAME"))
    cls = conv2d(rpn, W_cls, b_cls, "VALID")
    reg = conv2d(rpn, W_reg, b_reg, "VALID")
    score_flat = jax.nn.sigmoid(
        jnp.transpose(cls, (0, 2, 3, 1)).reshape(-1))
    deltas = jnp.transpose(reg[0].reshape(_A * 4, _P))
    deltas = jnp.pad(deltas, ((0, 0), (0, 64 - 4 * _A)))

    sel = np.zeros((64, 64), np.float32)
    for a in range(_A):
        for c in range(4):
            sel[4 * a + c, 16 * c + a] = 1.0
    sel = jnp.asarray(sel)
    aw, ah, acx, acy = _anchor_planes(image.shape, feat.shape)
    plane = lambda: pl.BlockSpec((_MB, _AL), lambda i: (i, 0))
    dec = pl.pallas_call(
        functools.partial(_decode_kernel, float(image.shape[-2]),
                          float(image.shape[-1])),
        grid=(_P // _MB,),
        in_specs=[
            pl.BlockSpec((_MB, 64), lambda i: (i, 0)),
            plane(), plane(), plane(), plane(),
            pl.BlockSpec((64, 64), lambda i: (0, 0)),
        ],
        out_specs=pl.BlockSpec((_MB, 128), lambda i: (i, 0)),
        out_shape=jax.ShapeDtypeStruct((_P, 128), f32),
    )(deltas, aw, ah, acx, acy, sel)

    lanes = jnp.arange(_AL)
    av = lanes[None, :] < _A
    score9 = score_flat.reshape(_P, _A)
    score16 = jnp.concatenate([score9, jnp.full((_P, _AL - _A), -1.0, f32)],
                              axis=1)
    ok16 = dec[:, 0:16]
    scf16 = jnp.where((ok16 > 0.0) & av, score16, -1.0)

    top_sc, top_idx = lax.top_k(score16.reshape(-1), _K1)
    scg = scf16.reshape(-1)[top_idx]
    order = jnp.argsort(-scg)
    idx_s = top_idx[order]
    sc_s = scg[order]
    padn = _NPAD - _K1
    idx_pad = jnp.pad(idx_s, (0, padn)).astype(jnp.int32)
    mesh = plsc.VectorSubcoreMesh(core_axis_name="c", subcore_axis_name="s")
    rows = pl.kernel(
        _sc_gather_rows, mesh=mesh,
        out_type=jax.ShapeDtypeStruct((_NPAD, 128), f32),
        scratch_types=[
            pltpu.VMEM((_BPW,), jnp.int32),
            pltpu.VMEM((_BPW,), jnp.int32),
            pltpu.VMEM((_BPW, 128), f32),
            pltpu.SemaphoreType.DMA,
        ])(dec, idx_pad)
    a_col = (idx_pad & 15)[:, None]
    box4 = jnp.concatenate(
        [jnp.take_along_axis(rows, 16 + 16 * c + a_col, axis=1)
         for c in range(4)], axis=1).T
    nms_in = jnp.concatenate([
        box4,
        jnp.pad(sc_s[None, :], ((0, 0), (0, padn)), constant_values=-1.0),
        jnp.zeros((3, _NPAD), f32)], axis=0)

    boxt = jnp.pad(box4.T, ((0, 0), (0, 124)))
    res = pl.pallas_call(
        _nms_kernel,
        out_shape=jax.ShapeDtypeStruct((_TOPK + 4, 128), f32),
    )(nms_in, boxt)
    return res[:_TOPK, 0:4], res[:_TOPK, 4]
